# Initial kernel scaffold; baseline (speedup 1.0000x reference)
import jax
import jax.numpy as jnp
from jax.experimental import pallas as pl


def kernel(preds, input, height, width, warp_matrix):
    raise NotImplementedError("write your pallas kernel here")


# speedup vs baseline: 6.6583x; 6.6583x over previous
import jax
import jax.numpy as jnp
from jax import lax
from jax.experimental import pallas as pl
from jax.experimental.pallas import tpu as pltpu
from jax.experimental.pallas import tpu_sc as plsc

_CONF_THRESH = 0.35
_IOU_THRESH = 0.6
_NMS_MAX_NUM = 100
_NUM_CLASSES = 80
_N_BOXES = 20000
_NPAD = 20480
_ROWS = _NPAD // 128

_NS = 16
_PERW = _NPAD // _NS
_CH = _PERW // 16

_NEG_INF = float("-inf")


def _decode_body(prT_ref, h_ref, w_ref,
                 x1_ref, y1_ref, x2_ref, y2_ref,
                 nx1_ref, ny1_ref, nx2_ref, ny2_ref,
                 ar_ref, s0_ref, lb_ref):
    W = w_ref[0, 0]
    H = h_ref[0, 0]
    mlog = prT_ref[4]
    lab = jnp.zeros((_ROWS, 128), jnp.float32)
    for c in range(1, _NUM_CLASSES):
        v = prT_ref[4 + c]
        upd = v > mlog
        mlog = jnp.where(upd, v, mlog)
        lab = jnp.where(upd, jnp.float32(c), lab)
    scores = jax.nn.sigmoid(mlog)

    cx = jax.nn.sigmoid(prT_ref[0]) * W
    cy = jax.nn.sigmoid(prT_ref[1]) * H
    bw = jax.nn.sigmoid(prT_ref[2]) * W * 0.3
    bh = jax.nn.sigmoid(prT_ref[3]) * H * 0.3
    x1 = cx - bw / 2.0
    y1 = cy - bh / 2.0
    x2 = cx + bw / 2.0
    y2 = cy + bh / 2.0

    ridx = jax.lax.broadcasted_iota(jnp.int32, (_ROWS, 128), 0)
    cidx = jax.lax.broadcasted_iota(jnp.int32, (_ROWS, 128), 1)
    idx = ridx * 128 + cidx
    valid_n = idx < _N_BOXES
    zero = jnp.zeros((_ROWS, 128), jnp.float32)
    x1 = jnp.where(valid_n, x1, zero)
    y1 = jnp.where(valid_n, y1, zero)
    x2 = jnp.where(valid_n, x2, zero)
    y2 = jnp.where(valid_n, y2, zero)
    lab = jnp.where(valid_n, lab, zero)
    s0 = jnp.where(valid_n & (scores > _CONF_THRESH), scores,
                   jnp.float32(_NEG_INF))

    max_coord = jnp.max(jnp.maximum(jnp.maximum(x1, y1), jnp.maximum(x2, y2)))
    off = lab * (max_coord + 1.0)
    nx1 = x1 + off
    ny1 = y1 + off
    nx2 = x2 + off
    ny2 = y2 + off
    areas = jnp.maximum(nx2 - nx1, 0.0) * jnp.maximum(ny2 - ny1, 0.0)

    x1_ref[...] = x1
    y1_ref[...] = y1
    x2_ref[...] = x2
    y2_ref[...] = y2
    nx1_ref[...] = nx1
    ny1_ref[...] = ny1
    nx2_ref[...] = nx2
    ny2_ref[...] = ny2
    ar_ref[...] = areas
    s0_ref[...] = s0
    lb_ref[...] = lab


def _sc_nms_body(s_hbm, nx1_hbm, ny1_hbm, nx2_hbm, ny2_hbm, ar_hbm,
                 x1_hbm, y1_hbm, x2_hbm, y2_hbm, lb_hbm, out_hbm,
                 s_v, nx1_v, ny1_v, nx2_v, ny2_v, ar_v,
                 x1_v, y1_v, x2_v, y2_v, lb_v,
                 pub_v, all_v, acc_v, shared):
    wid = lax.axis_index("s")
    base = wid * _PERW
    sl = pl.ds(base, _PERW)
    pltpu.sync_copy(s_hbm.at[sl], s_v)
    pltpu.sync_copy(nx1_hbm.at[sl], nx1_v)
    pltpu.sync_copy(ny1_hbm.at[sl], ny1_v)
    pltpu.sync_copy(nx2_hbm.at[sl], nx2_v)
    pltpu.sync_copy(ny2_hbm.at[sl], ny2_v)
    pltpu.sync_copy(ar_hbm.at[sl], ar_v)
    pltpu.sync_copy(x1_hbm.at[sl], x1_v)
    pltpu.sync_copy(y1_hbm.at[sl], y1_v)
    pltpu.sync_copy(x2_hbm.at[sl], x2_v)
    pltpu.sync_copy(y2_hbm.at[sl], y2_v)
    pltpu.sync_copy(lb_hbm.at[sl], lb_v)

    iota16i = lax.iota(jnp.int32, 16)
    iota16 = iota16i.astype(jnp.float32)
    basef = jnp.float32(1.0) * base.astype(jnp.float32)
    neg = jnp.float32(_NEG_INF)
    big = jnp.float32(3e7)

    def sel(j, val, cur):
        return jnp.where(iota16i == j, val, cur)

    def step(i, carry):
        def amax(k, c2):
            vmax, vidx = c2
            v = s_v[pl.ds(k * 16, 16)]
            gidx = basef + (16.0 * k.astype(jnp.float32)) + iota16
            upd = v > vmax
            return (jnp.where(upd, v, vmax), jnp.where(upd, gidx, vidx))

        vmax = jnp.full((16,), neg, jnp.float32)
        vidx = jnp.full((16,), big, jnp.float32)
        for k in range(_CH):
            vmax, vidx = amax(jnp.int32(k), (vmax, vidx))
        m_loc = lax.reduce_max(vmax, axes=(0,))
        i_loc = lax.reduce_min(jnp.where(vmax == m_loc, vidx, big), axes=(0,))
        off = (i_loc - basef).astype(jnp.int32)
        offc = jnp.minimum(jnp.maximum(off, 0), _PERW - 1)
        idxv = jnp.full((16,), 0, jnp.int32) + offc

        def pk(ref):
            return plsc.load_gather(ref, [idxv])[0]

        pub = jnp.zeros((16,), jnp.float32)
        pub = sel(0, m_loc, pub)
        pub = sel(1, i_loc, pub)
        pub = sel(2, pk(nx1_v), pub)
        pub = sel(3, pk(ny1_v), pub)
        pub = sel(4, pk(nx2_v), pub)
        pub = sel(5, pk(ny2_v), pub)
        pub = sel(6, pk(x1_v), pub)
        pub = sel(7, pk(y1_v), pub)
        pub = sel(8, pk(x2_v), pub)
        pub = sel(9, pk(y2_v), pub)
        pub = sel(10, pk(lb_v), pub)
        pub_v[...] = pub
        pltpu.sync_copy(pub_v, shared.at[pl.ds(wid * 16, 16)])
        plsc.subcore_barrier()
        pltpu.sync_copy(shared, all_v)
        plsc.subcore_barrier()

        rbest = all_v[pl.ds(0, 16)]
        for w in range(1, _NS):
            rv = all_v[pl.ds(w * 16, 16)]
            better = (rv[0] > rbest[0]) | ((rv[0] == rbest[0])
                                           & (rv[1] < rbest[1]))
            rbest = jnp.where(better, rv, rbest)
        bm = rbest[0]
        bi = rbest[1]
        bnx1 = rbest[2]
        bny1 = rbest[3]
        bnx2 = rbest[4]
        bny2 = rbest[5]
        bx1 = rbest[6]
        by1 = rbest[7]
        bx2 = rbest[8]
        by2 = rbest[9]
        blb = rbest[10]
        validf = jnp.where(bm > neg, jnp.float32(1.0), jnp.float32(0.0))
        area1 = jnp.maximum(bnx2 - bnx1, 0.0) * jnp.maximum(bny2 - bny1, 0.0)

        def supp(k, _):
            sv = s_v[pl.ds(k * 16, 16)]
            cx1 = nx1_v[pl.ds(k * 16, 16)]
            cy1 = ny1_v[pl.ds(k * 16, 16)]
            cx2 = nx2_v[pl.ds(k * 16, 16)]
            cy2 = ny2_v[pl.ds(k * 16, 16)]
            car = ar_v[pl.ds(k * 16, 16)]
            ltx = jnp.maximum(bnx1, cx1)
            lty = jnp.maximum(bny1, cy1)
            rbx = jnp.minimum(bnx2, cx2)
            rby = jnp.minimum(bny2, cy2)
            ww = jnp.maximum(rbx - ltx, 0.0)
            hh = jnp.maximum(rby - lty, 0.0)
            inter = ww * hh
            iou = inter / (area1 + car - inter + 1e-7)
            gidx = basef + (16.0 * k.astype(jnp.float32)) + iota16
            kill = (iou > _IOU_THRESH) | (gidx == bi)
            s_v[pl.ds(k * 16, 16)] = jnp.where(kill, neg, sv)
            return 0

        for k in range(_CH):
            supp(jnp.int32(k), 0)

        @pl.when(wid == 0)
        def _():
            row = jnp.zeros((16,), jnp.float32)
            row = sel(0, bx1, row)
            row = sel(1, by1, row)
            row = sel(2, bx2, row)
            row = sel(3, by2, row)
            row = sel(4, bm, row)
            row = sel(5, blb, row)
            row = sel(6, validf, row)
            acc_v[pl.ds(i * 16, 16)] = row

        return carry

    lax.fori_loop(0, _NMS_MAX_NUM, step, 0)

    @pl.when(wid == 0)
    def _():
        pltpu.sync_copy(acc_v, out_hbm)


def _warp_body(acc_ref, h_ref, w_ref, warp_ref, out_ref):
    W = w_ref[0, 0]
    H = h_ref[0, 0]
    a = warp_ref[0, 0]
    b = warp_ref[0, 1]
    c = warp_ref[0, 2]
    d = warp_ref[1, 0]
    e = warp_ref[1, 1]
    f = warp_ref[1, 2]
    g = warp_ref[2, 0]
    h = warp_ref[2, 1]
    i_ = warp_ref[2, 2]
    det = a * (e * i_ - f * h) - b * (d * i_ - f * g) + c * (d * h - e * g)
    i00 = (e * i_ - f * h) / det
    i01 = (c * h - b * i_) / det
    i02 = (b * f - c * e) / det
    i10 = (f * g - d * i_) / det
    i11 = (a * i_ - c * g) / det
    i12 = (c * d - a * f) / det
    i20 = (d * h - e * g) / det
    i21 = (b * g - a * h) / det
    i22 = (a * e - b * d) / det

    A = acc_ref[...]
    kx1 = A[:, 0:1]
    ky1 = A[:, 1:2]
    kx2 = A[:, 2:3]
    ky2 = A[:, 3:4]
    ksc = A[:, 4:5]
    klb = A[:, 5:6]
    kvl = A[:, 6:7]

    def warp_pt(xs, ys):
        t0 = i00 * xs + i01 * ys + i02
        t1 = i10 * xs + i11 * ys + i12
        t2 = i20 * xs + i21 * ys + i22
        return t0 / (t2 + 1e-9), t1 / (t2 + 1e-9)

    xa, ya = warp_pt(kx1, ky1)
    xb, yb = warp_pt(kx2, ky1)
    xc, yc = warp_pt(kx1, ky2)
    xd, yd = warp_pt(kx2, ky2)
    wx1 = jnp.clip(jnp.minimum(jnp.minimum(xa, xb), jnp.minimum(xc, xd)), 0.0, W)
    wy1 = jnp.clip(jnp.minimum(jnp.minimum(ya, yb), jnp.minimum(yc, yd)), 0.0, H)
    wx2 = jnp.clip(jnp.maximum(jnp.maximum(xa, xb), jnp.maximum(xc, xd)), 0.0, W)
    wy2 = jnp.clip(jnp.maximum(jnp.maximum(ya, yb), jnp.maximum(yc, yd)), 0.0, H)

    dets = jnp.concatenate([wx1, wy1, wx2, wy2, ksc, klb], axis=1)
    dets = jnp.where(kvl > 0.5, dets, 0.0)
    out_ref[...] = dets


@jax.jit
def _run(preds, height, width, warp_matrix):
    pr = preds
    pr = jnp.pad(pr, ((0, _NPAD - _N_BOXES), (0, 0)))
    prT = jnp.transpose(pr).reshape(4 + _NUM_CLASSES, _ROWS, 128)
    h2 = jnp.reshape(height, (1, 1))
    w2 = jnp.reshape(width, (1, 1))

    shp = jax.ShapeDtypeStruct((_ROWS, 128), jnp.float32)
    dec = pl.pallas_call(
        _decode_body,
        out_shape=(shp,) * 11,
        in_specs=[
            pl.BlockSpec(memory_space=pltpu.VMEM),
            pl.BlockSpec(memory_space=pltpu.SMEM),
            pl.BlockSpec(memory_space=pltpu.SMEM),
        ],
        out_specs=(pl.BlockSpec(memory_space=pltpu.VMEM),) * 11,
    )(prT, h2, w2)
    x1, y1, x2, y2, nx1, ny1, nx2, ny2, ar, s0, lb = [
        v.reshape(_NPAD) for v in dec]

    mesh = plsc.VectorSubcoreMesh(core_axis_name="c", subcore_axis_name="s",
                                  num_cores=1)
    nms = pl.kernel(
        _sc_nms_body,
        mesh=mesh,
        compiler_params=pltpu.CompilerParams(needs_layout_passes=False),
        out_type=jax.ShapeDtypeStruct((_NMS_MAX_NUM * 16,), jnp.float32),
        scratch_types=(
            [pltpu.VMEM((_PERW,), jnp.float32) for _ in range(11)]
            + [pltpu.VMEM((16,), jnp.float32),
               pltpu.VMEM((_NS * 16,), jnp.float32),
               pltpu.VMEM((_NMS_MAX_NUM * 16,), jnp.float32),
               pltpu.VMEM_SHARED((_NS * 16,), jnp.float32)]
        ),
    )
    acc = nms(s0, nx1, ny1, nx2, ny2, ar, x1, y1, x2, y2, lb)
    acc2 = acc.reshape(_NMS_MAX_NUM, 16)

    return pl.pallas_call(
        _warp_body,
        out_shape=jax.ShapeDtypeStruct((_NMS_MAX_NUM, 6), jnp.float32),
        in_specs=[
            pl.BlockSpec(memory_space=pltpu.VMEM),
            pl.BlockSpec(memory_space=pltpu.SMEM),
            pl.BlockSpec(memory_space=pltpu.SMEM),
            pl.BlockSpec(memory_space=pltpu.SMEM),
        ],
        out_specs=pl.BlockSpec(memory_space=pltpu.VMEM),
    )(acc2, h2, w2, warp_matrix)


def kernel(preds, input, height, width, warp_matrix):
    del input
    return _run(preds, height, width, warp_matrix)


def _sc_run_nms(s0, nx1, ny1, nx2, ny2, ar, x1, y1, x2, y2, lb):
    mesh = plsc.VectorSubcoreMesh(core_axis_name="c", subcore_axis_name="s",
                                  num_cores=1)
    f = pl.kernel(
        _sc_nms_body,
        mesh=mesh,
        compiler_params=pltpu.CompilerParams(needs_layout_passes=False),
        out_type=jax.ShapeDtypeStruct((_NMS_MAX_NUM * 16,), jnp.float32),
        scratch_types=(
            [pltpu.VMEM((_PERW,), jnp.float32) for _ in range(11)]
            + [pltpu.VMEM((16,), jnp.float32),
               pltpu.VMEM((_NS * 16,), jnp.float32),
               pltpu.VMEM((_NMS_MAX_NUM * 16,), jnp.float32),
               pltpu.VMEM_SHARED((_NS * 16,), jnp.float32)]
        ),
    )
    return f(s0, nx1, ny1, nx2, ny2, ar, x1, y1, x2, y2, lb)
